# baseline (speedup 1.0000x reference)
import jax
import jax.numpy as jnp
from jax.experimental import pallas as pl
from jax.experimental.pallas import tpu as pltpu

BM = 1024
BN = 512

F8 = jnp.float8_e4m3fn
LO_SCALE = 16.0


def _split_f8(v):
    hi = v.astype(F8)
    lo = ((v - hi.astype(jnp.float32)) * LO_SCALE).astype(F8)
    return hi, lo


def _dot_nt(a, b_):
    return jax.lax.dot_general(
        a, b_,
        dimension_numbers=(((1,), (1,)), ((), ())),
        preferred_element_type=jnp.float32,
    )


def _mm_body(x_ref, w_ref, b_ref, o_ref, xh_ref, xl_ref):
    j = pl.program_id(1)

    @pl.when(j == 0)
    def _():
        xh, xl = _split_f8(x_ref[...])
        xh_ref[...] = xh
        xl_ref[...] = xl

    wh, wl = _split_f8(w_ref[...])
    xh = xh_ref[...]
    acc = _dot_nt(xh, wh)
    corr = _dot_nt(xl_ref[...], wh) + _dot_nt(xh, wl)
    o_ref[...] = acc + corr * (1.0 / LO_SCALE) + b_ref[...]


@jax.jit
def kernel(x, W, b):
    M, K = x.shape
    N = W.shape[0]
    b2 = b.reshape(1, N)
    out = pl.pallas_call(
        _mm_body,
        grid=(M // BM, N // BN),
        in_specs=[
            pl.BlockSpec((BM, K), lambda i, j: (i, 0)),
            pl.BlockSpec((BN, K), lambda i, j: (j, 0)),
            pl.BlockSpec((1, BN), lambda i, j: (0, j)),
        ],
        out_specs=pl.BlockSpec((BM, BN), lambda i, j: (i, j)),
        out_shape=jax.ShapeDtypeStruct((M, N), jnp.float32),
        scratch_shapes=[
            pltpu.VMEM((BM, K), F8),
            pltpu.VMEM((BM, K), F8),
        ],
        compiler_params=pltpu.CompilerParams(
            dimension_semantics=("parallel", "arbitrary"),
            vmem_limit_bytes=100 * 1024 * 1024,
        ),
    )(x, W, b2)
    return out


# speedup vs baseline: 1.3907x; 1.3907x over previous
import jax
import jax.numpy as jnp
from jax.experimental import pallas as pl
from jax.experimental.pallas import tpu as pltpu

BM = 2048
CK = 1024
BN = 256


def _mm_body(x_ref, w_ref, b_ref, o_ref, xb_ref):
    t = pl.program_id(1)
    nc = xb_ref.shape[1] // x_ref.shape[1]

    @pl.when(t < nc)
    def _():
        base = pl.multiple_of(t * CK, CK)
        xb_ref[:, pl.ds(base, CK)] = x_ref[...].astype(jnp.bfloat16)

    @pl.when(t >= nc)
    def _():
        acc = jax.lax.dot_general(
            xb_ref[...],
            w_ref[...].astype(jnp.bfloat16),
            dimension_numbers=(((1,), (1,)), ((), ())),
            preferred_element_type=jnp.float32,
        )
        o_ref[...] = acc + b_ref[...]


@jax.jit
def kernel(x, W, b):
    M, K = x.shape
    N = W.shape[0]
    nc = K // CK
    nd = N // BN
    b2 = b.reshape(1, N)
    out = pl.pallas_call(
        _mm_body,
        grid=(M // BM, nc + nd),
        in_specs=[
            pl.BlockSpec(
                (BM, CK), lambda i, t: (i, jnp.where(t < nc, t, nc - 1))
            ),
            pl.BlockSpec(
                (BN, K), lambda i, t: (jnp.where(t >= nc, t - nc, 0), 0)
            ),
            pl.BlockSpec(
                (1, BN), lambda i, t: (0, jnp.where(t >= nc, t - nc, 0))
            ),
        ],
        out_specs=pl.BlockSpec(
            (BM, BN), lambda i, t: (i, jnp.where(t >= nc, t - nc, 0))
        ),
        out_shape=jax.ShapeDtypeStruct((M, N), jnp.float32),
        scratch_shapes=[pltpu.VMEM((BM, K), jnp.bfloat16)],
        compiler_params=pltpu.CompilerParams(
            dimension_semantics=("arbitrary", "arbitrary"),
            vmem_limit_bytes=100 * 1024 * 1024,
        ),
    )(x, W, b2)
    return out


# speedup vs baseline: 1.4510x; 1.0434x over previous
import jax
import jax.numpy as jnp
from jax.experimental import pallas as pl
from jax.experimental.pallas import tpu as pltpu

BM = 2048
CK = 1024
BN = 512


def _mm_body(x_ref, w_ref, b_ref, o_ref, xb_ref):
    t = pl.program_id(1)
    nc = xb_ref.shape[1] // x_ref.shape[1]

    @pl.when(t < nc)
    def _():
        base = pl.multiple_of(t * CK, CK)
        xb_ref[:, pl.ds(base, CK)] = x_ref[...].astype(jnp.bfloat16)

    @pl.when(t >= nc)
    def _():
        acc = jax.lax.dot_general(
            xb_ref[...],
            w_ref[...].astype(jnp.bfloat16),
            dimension_numbers=(((1,), (1,)), ((), ())),
            preferred_element_type=jnp.float32,
        )
        o_ref[...] = acc + b_ref[...]


@jax.jit
def kernel(x, W, b):
    M, K = x.shape
    N = W.shape[0]
    nc = K // CK
    nd = N // BN
    b2 = b.reshape(1, N)
    out = pl.pallas_call(
        _mm_body,
        grid=(M // BM, nc + nd),
        in_specs=[
            pl.BlockSpec(
                (BM, CK), lambda i, t: (i, jnp.where(t < nc, t, nc - 1))
            ),
            pl.BlockSpec(
                (BN, K), lambda i, t: (jnp.where(t >= nc, t - nc, 0), 0)
            ),
            pl.BlockSpec(
                (1, BN), lambda i, t: (0, jnp.where(t >= nc, t - nc, 0))
            ),
        ],
        out_specs=pl.BlockSpec(
            (BM, BN), lambda i, t: (i, jnp.where(t >= nc, t - nc, 0))
        ),
        out_shape=jax.ShapeDtypeStruct((M, N), jnp.float32),
        scratch_shapes=[pltpu.VMEM((BM, K), jnp.bfloat16)],
        compiler_params=pltpu.CompilerParams(
            dimension_semantics=("arbitrary", "arbitrary"),
            vmem_limit_bytes=100 * 1024 * 1024,
        ),
    )(x, W, b2)
    return out
